# Initial kernel scaffold; baseline (speedup 1.0000x reference)
import jax
import jax.numpy as jnp
from jax.experimental import pallas as pl


def kernel(x, z, W_x2y, W_z2y, W_y2z, per_item):
    raise NotImplementedError("write your pallas kernel here")


# speedup vs baseline: 2.9895x; 2.9895x over previous
import functools

import jax
import jax.numpy as jnp
from jax import lax
from jax.experimental import pallas as pl
from jax.experimental.pallas import tpu as pltpu
from jax.experimental.pallas import tpu_sc as plsc

_YT = 1000

_NC, _NS, _L = 2, 16, 16
_NW = _NC * _NS


def _tc_body(x_ref, wx_ref, wz_ref, idx_ref, inv_ref,
             xn_ref, bv_ref, bi_ref, acc_ref, *, nsteps, yt):
    j = pl.program_id(0)

    @pl.when(j == 0)
    def _init():
        xv = x_ref[...]
        n = jnp.sqrt(jnp.sum(xv * xv, axis=1, keepdims=True))
        xn_ref[...] = (xv / jnp.maximum(n, 1e-12)).astype(jnp.bfloat16)
        bv_ref[...] = jnp.full(bv_ref.shape, -jnp.inf, jnp.float32)
        bi_ref[...] = jnp.zeros(bi_ref.shape, jnp.int32)

    w = wx_ref[...]
    n = jnp.sqrt(jnp.sum(w * w, axis=1, keepdims=True))
    wn = (w / jnp.maximum(n, 1e-12)).astype(jnp.bfloat16)
    scores = lax.dot_general(xn_ref[...], wn, (((1,), (1,)), ((), ())),
                             preferred_element_type=jnp.float32)
    m = jnp.max(scores, axis=1, keepdims=True)
    iota = lax.broadcasted_iota(jnp.int32, scores.shape, 1)
    lidx = jnp.min(jnp.where(scores == m, iota, yt), axis=1, keepdims=True)
    better = m > bv_ref[...]
    bi_ref[...] = jnp.where(better, lidx + j * yt, bi_ref[...])
    bv_ref[...] = jnp.where(better, m, bv_ref[...])

    wz = wz_ref[...]
    cps = wz.shape[0]
    acc_ref[pl.ds(j * cps, cps)] = jnp.sum(wz * wz, axis=1, keepdims=True)

    @pl.when(j == nsteps - 1)
    def _fin():
        idx_ref[...] = bi_ref[...]
        z = inv_ref.shape[0]
        s = jnp.sum(acc_ref[...].reshape(z, 8), axis=1, keepdims=True)
        inv_ref[...] = 1.0 / jnp.maximum(jnp.sqrt(s), 1e-12)


def _tc_stage(x, W_x2y, W_y2z):
    B, D = x.shape
    Y = W_x2y.shape[0]
    Z = W_y2z.shape[0]
    nsteps = Y // _YT
    wz8 = W_y2z.reshape(8 * Z, Y // 8)
    cps = (8 * Z) // nsteps
    assert cps * nsteps == 8 * Z and cps % 8 == 0
    return pl.pallas_call(
        functools.partial(_tc_body, nsteps=nsteps, yt=_YT),
        grid=(nsteps,),
        in_specs=[
            pl.BlockSpec((B, D), lambda j: (0, 0)),
            pl.BlockSpec((_YT, D), lambda j: (j, 0)),
            pl.BlockSpec((cps, Y // 8), lambda j: (j, 0)),
        ],
        out_specs=[
            pl.BlockSpec((B, 1), lambda j: (0, 0)),
            pl.BlockSpec((Z, 1), lambda j: (0, 0)),
        ],
        out_shape=[
            jax.ShapeDtypeStruct((B, 1), jnp.int32),
            jax.ShapeDtypeStruct((Z, 1), jnp.float32),
        ],
        scratch_shapes=[
            pltpu.VMEM((B, D), jnp.bfloat16),
            pltpu.VMEM((B, 1), jnp.float32),
            pltpu.VMEM((B, 1), jnp.int32),
            pltpu.VMEM((8 * Z, 1), jnp.float32),
        ],
    )(x, W_x2y, wz8)


def _sc_body(wflat_ref, idx_ref, inv_ref, out_ref,
             idxv, zoff, fidx, vals, invv, sem, *, y, z, bpw):
    wid = lax.axis_index("s") * _NC + lax.axis_index("c")
    base = wid * bpw
    pltpu.sync_copy(idx_ref.at[pl.ds(base, bpw)], idxv)
    pltpu.sync_copy(inv_ref, invv)
    for k in range(128 // _L):
        col = lax.broadcasted_iota(jnp.int32, (_L,), 0) + (_L * k)
        zoff[pl.ds(_L * k, _L)] = jnp.where(col < z, col * y, 0)
    chunks = [idxv[pl.ds(c * _L, _L)] for c in range(bpw // _L)]
    for b in range(bpw):
        bidx = chunks[b // _L][b % _L]
        for k in range(128 // _L):
            s = pl.ds(_L * k, _L)
            fidx[b, s] = zoff[s] + bidx
    copies = [
        pltpu.async_copy(wflat_ref.at[fidx.at[b]], vals.at[b], sem)
        for b in range(bpw)
    ]
    for c in copies:
        c.wait()
    for b in range(bpw):
        for k in range(128 // _L):
            s = pl.ds(_L * k, _L)
            vals[b, s] = vals[b, s] * invv[s]
    pltpu.sync_copy(vals, out_ref.at[pl.ds(base, bpw)])


def _sc_stage(wflat, idx, inv128, B, Y, Z):
    bpw = B // _NW
    mesh = plsc.VectorSubcoreMesh(core_axis_name="c", subcore_axis_name="s")
    k = pl.kernel(
        functools.partial(_sc_body, y=Y, z=Z, bpw=bpw),
        out_type=jax.ShapeDtypeStruct((B, 128), jnp.float32),
        mesh=mesh,
        scratch_types=[
            pltpu.VMEM((bpw,), jnp.int32),
            pltpu.VMEM((128,), jnp.int32),
            pltpu.VMEM((bpw, 128), jnp.int32),
            pltpu.VMEM((bpw, 128), jnp.float32),
            pltpu.VMEM((128,), jnp.float32),
            pltpu.SemaphoreType.DMA,
        ],
    )
    return k(wflat, idx, inv128)


def kernel(x, z, W_x2y, W_z2y, W_y2z, per_item):
    B, D = x.shape
    Y = W_x2y.shape[0]
    Z = W_y2z.shape[0]
    idx2d, inv2d = _tc_stage(x, W_x2y, W_y2z)
    idx = idx2d.reshape(B)
    inv128 = jnp.pad(inv2d.reshape(Z), (0, 128 - Z))
    wflat = W_y2z.reshape(Z * Y)
    outp = _sc_stage(wflat, idx, inv128, B, Y, Z)
    out = outp[:, :Z]
    return jnp.where(per_item >= 1, out, jnp.zeros_like(out))


# speedup vs baseline: 3.4639x; 1.1587x over previous
import functools

import jax
import jax.numpy as jnp
from jax import lax
from jax.experimental import pallas as pl
from jax.experimental.pallas import tpu as pltpu
from jax.experimental.pallas import tpu_sc as plsc

_YT = 2000

_NC, _NS, _L = 2, 16, 16
_NW = _NC * _NS


def _tc_body(x_ref, wx_ref, wz_ref, idx_ref, inv_ref,
             xn_ref, bv_ref, bi_ref, acc_ref, io_ref, *, nsteps, yt):
    j = pl.program_id(0)

    @pl.when(j == 0)
    def _init():
        xv = x_ref[...]
        n = jnp.sqrt(jnp.sum(xv * xv, axis=1, keepdims=True))
        xn_ref[...] = (xv / jnp.maximum(n, 1e-12)).astype(jnp.bfloat16)
        bv_ref[...] = jnp.full(bv_ref.shape, -jnp.inf, jnp.float32)
        bi_ref[...] = jnp.zeros(bi_ref.shape, jnp.float32)
        io_ref[...] = lax.broadcasted_iota(
            jnp.int32, io_ref.shape, 1).astype(jnp.float32)

    w = wx_ref[...]
    n = jnp.sqrt(jnp.sum(w * w, axis=1, keepdims=True))
    wn = (w / jnp.maximum(n, 1e-12)).astype(jnp.bfloat16)
    scores = lax.dot_general(xn_ref[...], wn, (((1,), (1,)), ((), ())),
                             preferred_element_type=jnp.float32)
    m = jnp.max(scores, axis=1, keepdims=True)
    iota = jnp.broadcast_to(io_ref[0:1, :], scores.shape)
    lidx = jnp.min(jnp.where(scores == m, iota, float(yt)), axis=1,
                   keepdims=True)
    better = m > bv_ref[...]
    bi_ref[...] = jnp.where(better, lidx + j * float(yt), bi_ref[...])
    bv_ref[...] = jnp.where(better, m, bv_ref[...])

    wz = wz_ref[...]
    cps = wz.shape[0]
    acc_ref[pl.ds(j * cps, cps)] = jnp.sum(wz * wz, axis=1, keepdims=True)

    @pl.when(j == nsteps - 1)
    def _fin():
        idx_ref[...] = bi_ref[...].astype(jnp.int32)
        z = inv_ref.shape[0]
        s = jnp.sum(acc_ref[...].reshape(z, 8), axis=1, keepdims=True)
        inv_ref[...] = 1.0 / jnp.maximum(jnp.sqrt(s), 1e-12)


def _tc_stage(x, W_x2y, W_y2z):
    B, D = x.shape
    Y = W_x2y.shape[0]
    Z = W_y2z.shape[0]
    nsteps = Y // _YT
    wz8 = W_y2z.reshape(8 * Z, Y // 8)
    cps = (8 * Z) // nsteps
    assert cps * nsteps == 8 * Z and cps % 8 == 0
    return pl.pallas_call(
        functools.partial(_tc_body, nsteps=nsteps, yt=_YT),
        grid=(nsteps,),
        in_specs=[
            pl.BlockSpec((B, D), lambda j: (0, 0)),
            pl.BlockSpec((_YT, D), lambda j: (j, 0)),
            pl.BlockSpec((cps, Y // 8), lambda j: (j, 0)),
        ],
        out_specs=[
            pl.BlockSpec((B, 1), lambda j: (0, 0)),
            pl.BlockSpec((Z, 1), lambda j: (0, 0)),
        ],
        out_shape=[
            jax.ShapeDtypeStruct((B, 1), jnp.int32),
            jax.ShapeDtypeStruct((Z, 1), jnp.float32),
        ],
        scratch_shapes=[
            pltpu.VMEM((B, D), jnp.bfloat16),
            pltpu.VMEM((B, 1), jnp.float32),
            pltpu.VMEM((B, 1), jnp.float32),
            pltpu.VMEM((8 * Z, 1), jnp.float32),
            pltpu.VMEM((8, _YT), jnp.float32),
        ],
    )(x, W_x2y, wz8)


def _sc_body(wflat_ref, idx_ref, inv_ref, out_ref,
             idxv, zoff, fidx, vals, invv, sem, *, y, z, bpw):
    wid = lax.axis_index("s") * _NC + lax.axis_index("c")
    base = wid * bpw
    pltpu.sync_copy(idx_ref.at[pl.ds(base, bpw)], idxv)
    pltpu.sync_copy(inv_ref, invv)
    for k in range(128 // _L):
        col = lax.broadcasted_iota(jnp.int32, (_L,), 0) + (_L * k)
        zoff[pl.ds(_L * k, _L)] = jnp.where(col < z, col * y, 0)
    chunks = [idxv[pl.ds(c * _L, _L)] for c in range(bpw // _L)]
    for b in range(bpw):
        bidx = chunks[b // _L][b % _L]
        for k in range(128 // _L):
            s = pl.ds(_L * k, _L)
            fidx[b, s] = zoff[s] + bidx
    copies = [
        pltpu.async_copy(wflat_ref.at[fidx.at[b]], vals.at[b], sem)
        for b in range(bpw)
    ]
    for c in copies:
        c.wait()
    for b in range(bpw):
        for k in range(128 // _L):
            s = pl.ds(_L * k, _L)
            vals[b, s] = vals[b, s] * invv[s]
    pltpu.sync_copy(vals, out_ref.at[pl.ds(base, bpw)])


def _sc_stage(wflat, idx, inv128, B, Y, Z):
    bpw = B // _NW
    mesh = plsc.VectorSubcoreMesh(core_axis_name="c", subcore_axis_name="s")
    k = pl.kernel(
        functools.partial(_sc_body, y=Y, z=Z, bpw=bpw),
        out_type=jax.ShapeDtypeStruct((B, 128), jnp.float32),
        mesh=mesh,
        scratch_types=[
            pltpu.VMEM((bpw,), jnp.int32),
            pltpu.VMEM((128,), jnp.int32),
            pltpu.VMEM((bpw, 128), jnp.int32),
            pltpu.VMEM((bpw, 128), jnp.float32),
            pltpu.VMEM((128,), jnp.float32),
            pltpu.SemaphoreType.DMA,
        ],
    )
    return k(wflat, idx, inv128)


def kernel(x, z, W_x2y, W_z2y, W_y2z, per_item):
    B, D = x.shape
    Y = W_x2y.shape[0]
    Z = W_y2z.shape[0]
    idx2d, inv2d = _tc_stage(x, W_x2y, W_y2z)
    idx = idx2d.reshape(B)
    inv128 = jnp.pad(inv2d.reshape(Z), (0, 128 - Z))
    wflat = W_y2z.reshape(Z * Y)
    outp = _sc_stage(wflat, idx, inv128, B, Y, Z)
    out = outp[:, :Z]
    return jnp.where(per_item >= 1, out, jnp.zeros_like(out))


# speedup vs baseline: 4.1645x; 1.2023x over previous
import functools

import jax
import jax.numpy as jnp
from jax import lax
from jax.experimental import pallas as pl
from jax.experimental.pallas import tpu as pltpu
from jax.experimental.pallas import tpu_sc as plsc

_YT = 2000

_NC, _NS, _L = 2, 16, 16
_NW = _NC * _NS


def _tc_body(x_ref, wx_ref, idx_ref, xn_ref, bv_ref, bi_ref, io_ref,
             *, nsteps, yt):
    j = pl.program_id(0)

    @pl.when(j == 0)
    def _init():
        xv = x_ref[...]
        n = jnp.sqrt(jnp.sum(xv * xv, axis=1, keepdims=True))
        xn_ref[...] = (xv / jnp.maximum(n, 1e-12)).astype(jnp.bfloat16)
        bv_ref[...] = jnp.full(bv_ref.shape, -jnp.inf, jnp.float32)
        bi_ref[...] = jnp.zeros(bi_ref.shape, jnp.float32)
        io_ref[...] = lax.broadcasted_iota(
            jnp.int32, io_ref.shape, 1).astype(jnp.float32)

    w = wx_ref[...]
    n = jnp.sqrt(jnp.sum(w * w, axis=1, keepdims=True))
    wn = (w / jnp.maximum(n, 1e-12)).astype(jnp.bfloat16)
    scores = lax.dot_general(xn_ref[...], wn, (((1,), (1,)), ((), ())),
                             preferred_element_type=jnp.float32)
    m = jnp.max(scores, axis=1, keepdims=True)
    iota = jnp.broadcast_to(io_ref[0:1, :], scores.shape)
    lidx = jnp.min(jnp.where(scores == m, iota, float(yt)), axis=1,
                   keepdims=True)
    better = m > bv_ref[...]
    bi_ref[...] = jnp.where(better, lidx + j * float(yt), bi_ref[...])
    bv_ref[...] = jnp.where(better, m, bv_ref[...])

    @pl.when(j == nsteps - 1)
    def _fin():
        idx_ref[...] = bi_ref[...].astype(jnp.int32)


def _tc_stage(x, W_x2y):
    B, D = x.shape
    Y = W_x2y.shape[0]
    nsteps = Y // _YT
    return pl.pallas_call(
        functools.partial(_tc_body, nsteps=nsteps, yt=_YT),
        grid=(nsteps,),
        in_specs=[
            pl.BlockSpec((B, D), lambda j: (0, 0)),
            pl.BlockSpec((_YT, D), lambda j: (j, 0)),
        ],
        out_specs=pl.BlockSpec((B, 1), lambda j: (0, 0)),
        out_shape=jax.ShapeDtypeStruct((B, 1), jnp.int32),
        scratch_shapes=[
            pltpu.VMEM((B, D), jnp.bfloat16),
            pltpu.VMEM((B, 1), jnp.float32),
            pltpu.VMEM((B, 1), jnp.float32),
            pltpu.VMEM((8, _YT), jnp.float32),
        ],
    )(x, W_x2y)


def _lane_shuffle(v, idx):
    return lax.gather(
        v, idx.reshape(_L, 1),
        dimension_numbers=lax.GatherDimensionNumbers(
            offset_dims=(), collapsed_slice_dims=(0,), start_index_map=(0,)),
        slice_sizes=(1,), mode=lax.GatherScatterMode.PROMISE_IN_BOUNDS)


def _lane_sum(v):
    lane = lax.broadcasted_iota(jnp.int32, (_L,), 0)
    for step in (1, 2, 4, 8):
        v = v + _lane_shuffle(v, jnp.bitwise_xor(lane, step))
    return v


def _nr_rsqrt(s):
    xs = jnp.full((_L,), 129.0, jnp.float32)
    for _ in range(8):
        xs = 0.5 * (xs + s / xs)
    return 1.0 / jnp.maximum(xs, 1e-12)


_ROWS_PER_SUB = 4
_PIECE = 10000
_UNROLL = 8


def _scn_body(wflat_ref, invp_ref, buf, outv, sem, *, y, z):
    wid = lax.axis_index("s") * _NC + lax.axis_index("c")
    if True:
        inv_vec = jnp.zeros((_L,), jnp.float32)
        lane = lax.broadcasted_iota(jnp.int32, (_L,), 0)
        for r in range(_ROWS_PER_SUB):
            row = jnp.minimum(wid * _ROWS_PER_SUB + r, z - 1)
            accs = (jnp.zeros((_L,), jnp.float32),) * _UNROLL
            for p in range(y // _PIECE):
                pltpu.sync_copy(
                    wflat_ref.at[pl.ds(row * y + p * _PIECE, _PIECE)], buf)
                blk = _UNROLL * _L
                nblk = _PIECE // blk

                def body(i, a):
                    base = i * blk
                    new = []
                    for k in range(_UNROLL):
                        v = buf[pl.ds(base + k * _L, _L)]
                        new.append(a[k] + v * v)
                    return tuple(new)

                accs = lax.fori_loop(0, nblk, body, accs)
                tail = buf[pl.ds(nblk * blk, _L)]
                accs = (accs[0] + tail * tail,) + accs[1:]
            tot = accs[0]
            for k in range(1, _UNROLL):
                tot = tot + accs[k]
            s_splat = _lane_sum(tot)
            inv_vec = jnp.where(lane == r, s_splat, inv_vec)
        outv[...] = _nr_rsqrt(inv_vec)
        pltpu.sync_copy(outv, invp_ref.at[wid, pl.ds(0, _L)])


def _scn_stage(wflat, Y, Z):
    mesh = plsc.VectorSubcoreMesh(core_axis_name="c", subcore_axis_name="s")
    k = pl.kernel(
        functools.partial(_scn_body, y=Y, z=Z),
        out_type=jax.ShapeDtypeStruct((_NW, 128), jnp.float32),
        mesh=mesh,
        scratch_types=[
            pltpu.VMEM((_PIECE,), jnp.float32),
            pltpu.VMEM((_L,), jnp.float32),
            pltpu.SemaphoreType.DMA,
        ],
    )
    return k(wflat)


def _scg_body(wflat_ref, idx_ref, invp_ref, out_ref,
              idxv, zoff, fidx, vals, invp_v, invv, sem, *, y, z, bpw):
    wid = lax.axis_index("s") * _NC + lax.axis_index("c")
    base = wid * bpw
    pltpu.sync_copy(idx_ref.at[pl.ds(base, bpw)], idxv)
    pltpu.sync_copy(invp_ref, invp_v)
    lane = lax.broadcasted_iota(jnp.int32, (_L,), 0)
    lm4 = jnp.bitwise_and(lane, 3)
    lg4 = lax.shift_right_logical(lane, 2)
    for k in range(128 // _L):
        acc = jnp.zeros((_L,), jnp.float32)
        for i in range(4):
            r = invp_v[4 * k + i, pl.ds(0, _L)]
            acc = jnp.where(lg4 == i, _lane_shuffle(r, lm4), acc)
        invv[pl.ds(_L * k, _L)] = acc
    for k in range(128 // _L):
        col = lax.broadcasted_iota(jnp.int32, (_L,), 0) + (_L * k)
        zoff[pl.ds(_L * k, _L)] = jnp.where(col < z, col * y, 0)
    chunks = [idxv[pl.ds(c * _L, _L)] for c in range(bpw // _L)]
    for b in range(bpw):
        bidx = chunks[b // _L][b % _L]
        for k in range(128 // _L):
            s = pl.ds(_L * k, _L)
            fidx[b, s] = zoff[s] + bidx
    copies = [
        pltpu.async_copy(wflat_ref.at[fidx.at[b]], vals.at[b], sem)
        for b in range(bpw)
    ]
    for c in copies:
        c.wait()
    for b in range(bpw):
        for k in range(128 // _L):
            s = pl.ds(_L * k, _L)
            vals[b, s] = vals[b, s] * invv[s]
    pltpu.sync_copy(vals, out_ref.at[pl.ds(base, bpw)])


def _scg_stage(wflat, idx, invp, B, Y, Z):
    bpw = B // _NW
    mesh = plsc.VectorSubcoreMesh(core_axis_name="c", subcore_axis_name="s")
    k = pl.kernel(
        functools.partial(_scg_body, y=Y, z=Z, bpw=bpw),
        out_type=jax.ShapeDtypeStruct((B, 128), jnp.float32),
        mesh=mesh,
        scratch_types=[
            pltpu.VMEM((bpw,), jnp.int32),
            pltpu.VMEM((128,), jnp.int32),
            pltpu.VMEM((bpw, 128), jnp.int32),
            pltpu.VMEM((bpw, 128), jnp.float32),
            pltpu.VMEM((_NW, 128), jnp.float32),
            pltpu.VMEM((128,), jnp.float32),
            pltpu.SemaphoreType.DMA,
        ],
    )
    return k(wflat, idx, invp)


def kernel(x, z, W_x2y, W_z2y, W_y2z, per_item):
    B, D = x.shape
    Y = W_x2y.shape[0]
    Z = W_y2z.shape[0]
    wflat = W_y2z.reshape(Z * Y)
    idx2d = _tc_stage(x, W_x2y)
    invp = _scn_stage(wflat, Y, Z)
    idx = idx2d.reshape(B)
    outp = _scg_stage(wflat, idx, invp, B, Y, Z)
    out = outp[:, :Z]
    return jnp.where(per_item >= 1, out, jnp.zeros_like(out))


# speedup vs baseline: 5.1863x; 1.2453x over previous
import functools

import jax
import jax.numpy as jnp
from jax import lax
from jax.experimental import pallas as pl
from jax.experimental.pallas import tpu as pltpu
from jax.experimental.pallas import tpu_sc as plsc

_YT = 2000

_NC, _NS, _L = 2, 16, 16
_NW = _NC * _NS


def _tc_body(x_ref, wx_ref, idx_ref, xn_ref, bv_ref, bi_ref, io_ref,
             *, nsteps, yt):
    j = pl.program_id(0)

    @pl.when(j == 0)
    def _init():
        xv = x_ref[...]
        n = jnp.sqrt(jnp.sum(xv * xv, axis=1, keepdims=True))
        xn_ref[...] = (xv / jnp.maximum(n, 1e-12)).astype(jnp.bfloat16)
        bv_ref[...] = jnp.full(bv_ref.shape, -jnp.inf, jnp.float32)
        bi_ref[...] = jnp.zeros(bi_ref.shape, jnp.float32)
        io_ref[...] = lax.broadcasted_iota(
            jnp.int32, io_ref.shape, 1).astype(jnp.float32)

    w = wx_ref[...]
    n = jnp.sqrt(jnp.sum(w * w, axis=1, keepdims=True))
    wn = (w / jnp.maximum(n, 1e-12)).astype(jnp.bfloat16)
    scores = lax.dot_general(xn_ref[...], wn, (((1,), (1,)), ((), ())),
                             preferred_element_type=jnp.float32)
    m = jnp.max(scores, axis=1, keepdims=True)
    iota = jnp.broadcast_to(io_ref[0:1, :], scores.shape)
    lidx = jnp.min(jnp.where(scores == m, iota, float(yt)), axis=1,
                   keepdims=True)
    better = m > bv_ref[...]
    bi_ref[...] = jnp.where(better, lidx + j * float(yt), bi_ref[...])
    bv_ref[...] = jnp.where(better, m, bv_ref[...])

    @pl.when(j == nsteps - 1)
    def _fin():
        idx_ref[...] = bi_ref[...].astype(jnp.int32)


def _tc_stage(x, W_x2y):
    B, D = x.shape
    Y = W_x2y.shape[0]
    nsteps = Y // _YT
    return pl.pallas_call(
        functools.partial(_tc_body, nsteps=nsteps, yt=_YT),
        grid=(nsteps,),
        in_specs=[
            pl.BlockSpec((B, D), lambda j: (0, 0)),
            pl.BlockSpec((_YT, D), lambda j: (j, 0)),
        ],
        out_specs=pl.BlockSpec((B, 1), lambda j: (0, 0)),
        out_shape=jax.ShapeDtypeStruct((B, 1), jnp.int32),
        scratch_shapes=[
            pltpu.VMEM((B, D), jnp.bfloat16),
            pltpu.VMEM((B, 1), jnp.float32),
            pltpu.VMEM((B, 1), jnp.float32),
            pltpu.VMEM((8, _YT), jnp.float32),
        ],
    )(x, W_x2y)


def _lane_shuffle(v, idx):
    return lax.gather(
        v, idx.reshape(_L, 1),
        dimension_numbers=lax.GatherDimensionNumbers(
            offset_dims=(), collapsed_slice_dims=(0,), start_index_map=(0,)),
        slice_sizes=(1,), mode=lax.GatherScatterMode.PROMISE_IN_BOUNDS)


def _lane_sum(v):
    lane = lax.broadcasted_iota(jnp.int32, (_L,), 0)
    for step in (1, 2, 4, 8):
        v = v + _lane_shuffle(v, jnp.bitwise_xor(lane, step))
    return v


def _nr_rsqrt(s):
    xs = jnp.full((_L,), 129.0, jnp.float32)
    for _ in range(8):
        xs = 0.5 * (xs + s / xs)
    return 1.0 / jnp.maximum(xs, 1e-12)


_ROWS_PER_SUB = 4
_UNROLL = 8


def _scn_body(wz_ref, invp_ref, wflat_ref, buf, outv, sem, wsem, *, y, z):
    wid = lax.axis_index("s") * _NC + lax.axis_index("c")
    inv_vec = jnp.zeros((_L,), jnp.float32)
    lane = lax.broadcasted_iota(jnp.int32, (_L,), 0)
    for r in range(_ROWS_PER_SUB):
        row = jnp.minimum(wid * _ROWS_PER_SUB + r, z - 1)
        pltpu.sync_copy(wz_ref.at[row], buf)
        wcopy = pltpu.async_copy(buf, wflat_ref.at[pl.ds(row * y, y)], wsem)
        blk = _UNROLL * _L
        nblk = y // blk
        accs = (jnp.zeros((_L,), jnp.float32),) * _UNROLL

        def body(i, a):
            base = i * blk
            new = []
            for k in range(_UNROLL):
                v = buf[pl.ds(base + k * _L, _L)]
                new.append(a[k] + v * v)
            return tuple(new)

        accs = lax.fori_loop(0, nblk, body, accs)
        ntail = (y - nblk * blk) // _L
        new = list(accs)
        for k in range(ntail):
            v = buf[pl.ds(nblk * blk + k * _L, _L)]
            new[k] = new[k] + v * v
        tot = new[0]
        for k in range(1, _UNROLL):
            tot = tot + new[k]
        s_splat = _lane_sum(tot)
        inv_vec = jnp.where(lane == r, s_splat, inv_vec)
        wcopy.wait()
    outv[...] = _nr_rsqrt(inv_vec)
    pltpu.sync_copy(outv, invp_ref.at[wid, pl.ds(0, _L)])


def _scn_stage(W_y2z):
    Z, Y = W_y2z.shape
    mesh = plsc.VectorSubcoreMesh(core_axis_name="c", subcore_axis_name="s")
    k = pl.kernel(
        functools.partial(_scn_body, y=Y, z=Z),
        out_type=[
            jax.ShapeDtypeStruct((_NW, 128), jnp.float32),
            jax.ShapeDtypeStruct((Z * Y,), jnp.float32),
        ],
        mesh=mesh,
        compiler_params=pltpu.CompilerParams(use_tc_tiling_on_sc=True),
        scratch_types=[
            pltpu.VMEM((Y,), jnp.float32),
            pltpu.VMEM((_L,), jnp.float32),
            pltpu.SemaphoreType.DMA,
            pltpu.SemaphoreType.DMA,
        ],
    )
    return k(W_y2z)


def _scg_body(wflat_ref, idx_ref, invp_ref, out_ref,
              idxv, zoff, fidx, vals, invp_v, invv, sem, *, y, z, bpw):
    wid = lax.axis_index("s") * _NC + lax.axis_index("c")
    base = wid * bpw
    pltpu.sync_copy(idx_ref.at[pl.ds(base, bpw)], idxv)
    pltpu.sync_copy(invp_ref, invp_v)
    lane = lax.broadcasted_iota(jnp.int32, (_L,), 0)
    lm4 = jnp.bitwise_and(lane, 3)
    lg4 = lax.shift_right_logical(lane, 2)
    for k in range(128 // _L):
        acc = jnp.zeros((_L,), jnp.float32)
        for i in range(4):
            r = invp_v[4 * k + i, pl.ds(0, _L)]
            acc = jnp.where(lg4 == i, _lane_shuffle(r, lm4), acc)
        invv[pl.ds(_L * k, _L)] = acc
    for k in range(128 // _L):
        col = lax.broadcasted_iota(jnp.int32, (_L,), 0) + (_L * k)
        zoff[pl.ds(_L * k, _L)] = jnp.where(col < z, col * y, 0)
    chunks = [idxv[pl.ds(c * _L, _L)] for c in range(bpw // _L)]
    for b in range(bpw):
        bidx = chunks[b // _L][b % _L]
        for k in range(128 // _L):
            s = pl.ds(_L * k, _L)
            fidx[b, s] = zoff[s] + bidx
    copies = [
        pltpu.async_copy(wflat_ref.at[fidx.at[b]], vals.at[b], sem)
        for b in range(bpw)
    ]
    for c in copies:
        c.wait()
    for b in range(bpw):
        for k in range(128 // _L):
            s = pl.ds(_L * k, _L)
            vals[b, s] = vals[b, s] * invv[s]
    pltpu.sync_copy(vals, out_ref.at[pl.ds(base, bpw)])


def _scg_stage(wflat, idx, invp, B, Y, Z):
    bpw = B // _NW
    mesh = plsc.VectorSubcoreMesh(core_axis_name="c", subcore_axis_name="s")
    k = pl.kernel(
        functools.partial(_scg_body, y=Y, z=Z, bpw=bpw),
        out_type=jax.ShapeDtypeStruct((B, 128), jnp.float32),
        mesh=mesh,
        compiler_params=pltpu.CompilerParams(use_tc_tiling_on_sc=True),
        scratch_types=[
            pltpu.VMEM((bpw,), jnp.int32),
            pltpu.VMEM((128,), jnp.int32),
            pltpu.VMEM((bpw, 128), jnp.int32),
            pltpu.VMEM((bpw, 128), jnp.float32),
            pltpu.VMEM((_NW, 128), jnp.float32),
            pltpu.VMEM((128,), jnp.float32),
            pltpu.SemaphoreType.DMA,
        ],
    )
    return k(wflat, idx, invp)


def kernel(x, z, W_x2y, W_z2y, W_y2z, per_item):
    B, D = x.shape
    Y = W_x2y.shape[0]
    Z = W_y2z.shape[0]
    idx2d = _tc_stage(x, W_x2y)
    invp, wflat = _scn_stage(W_y2z)
    idx = idx2d.reshape(B)
    outp = _scg_stage(wflat, idx, invp, B, Y, Z)
    out = outp[:, :Z]
    return jnp.where(per_item >= 1, out, jnp.zeros_like(out))


# speedup vs baseline: 5.6512x; 1.0896x over previous
import functools

import jax
import jax.numpy as jnp
from jax import lax
from jax.experimental import pallas as pl
from jax.experimental.pallas import tpu as pltpu
from jax.experimental.pallas import tpu_sc as plsc

_YT = 5000

_NC, _NS, _L = 2, 16, 16
_NW = _NC * _NS


def _tc_body(x_ref, wx_ref, idx_ref, xn_ref, bv_ref, bi_ref, io_ref,
             *, nsteps, yt):
    j = pl.program_id(0)

    @pl.when(j == 0)
    def _init():
        xv = x_ref[...]
        n = jnp.sqrt(jnp.sum(xv * xv, axis=1, keepdims=True))
        xn_ref[...] = (xv / jnp.maximum(n, 1e-12)).astype(jnp.bfloat16)
        bv_ref[...] = jnp.full(bv_ref.shape, -jnp.inf, jnp.float32)
        bi_ref[...] = jnp.zeros(bi_ref.shape, jnp.float32)
        io_ref[...] = lax.broadcasted_iota(
            jnp.int32, io_ref.shape, 1).astype(jnp.float32)

    w = wx_ref[...]
    n = jnp.sqrt(jnp.sum(w * w, axis=1, keepdims=True))
    wn = (w / jnp.maximum(n, 1e-12)).astype(jnp.bfloat16)
    scores = lax.dot_general(xn_ref[...], wn, (((1,), (1,)), ((), ())),
                             preferred_element_type=jnp.float32)
    m = jnp.max(scores, axis=1, keepdims=True)
    iota = jnp.broadcast_to(io_ref[0:1, :], scores.shape)
    lidx = jnp.min(jnp.where(scores == m, iota, float(yt)), axis=1,
                   keepdims=True)
    better = m > bv_ref[...]
    bi_ref[...] = jnp.where(better, lidx + j * float(yt), bi_ref[...])
    bv_ref[...] = jnp.where(better, m, bv_ref[...])

    @pl.when(j == nsteps - 1)
    def _fin():
        idx_ref[...] = bi_ref[...].astype(jnp.int32)


def _tc_stage(x, W_x2y):
    B, D = x.shape
    Y = W_x2y.shape[0]
    nsteps = Y // _YT
    return pl.pallas_call(
        functools.partial(_tc_body, nsteps=nsteps, yt=_YT),
        grid=(nsteps,),
        in_specs=[
            pl.BlockSpec((B, D), lambda j: (0, 0)),
            pl.BlockSpec((_YT, D), lambda j: (j, 0)),
        ],
        out_specs=pl.BlockSpec((B, 1), lambda j: (0, 0)),
        out_shape=jax.ShapeDtypeStruct((B, 1), jnp.int32),
        scratch_shapes=[
            pltpu.VMEM((B, D), jnp.bfloat16),
            pltpu.VMEM((B, 1), jnp.float32),
            pltpu.VMEM((B, 1), jnp.float32),
            pltpu.VMEM((8, _YT), jnp.float32),
        ],
    )(x, W_x2y)


def _lane_shuffle(v, idx):
    return lax.gather(
        v, idx.reshape(_L, 1),
        dimension_numbers=lax.GatherDimensionNumbers(
            offset_dims=(), collapsed_slice_dims=(0,), start_index_map=(0,)),
        slice_sizes=(1,), mode=lax.GatherScatterMode.PROMISE_IN_BOUNDS)


def _lane_sum(v):
    lane = lax.broadcasted_iota(jnp.int32, (_L,), 0)
    for step in (1, 2, 4, 8):
        v = v + _lane_shuffle(v, jnp.bitwise_xor(lane, step))
    return v


def _nr_rsqrt(s):
    xs = jnp.full((_L,), 129.0, jnp.float32)
    for _ in range(8):
        xs = 0.5 * (xs + s / xs)
    return 1.0 / jnp.maximum(xs, 1e-12)


_ROWS_PER_SUB = 4
_UNROLL = 8


def _scn_body(wz_ref, invp_ref, wflat_ref, buf, outv, sem, wsem, *, y, z):
    wid = lax.axis_index("s") * _NC + lax.axis_index("c")
    inv_vec = jnp.zeros((_L,), jnp.float32)
    lane = lax.broadcasted_iota(jnp.int32, (_L,), 0)
    for r in range(_ROWS_PER_SUB):
        row = jnp.minimum(wid * _ROWS_PER_SUB + r, z - 1)
        pltpu.sync_copy(wz_ref.at[row], buf)
        wcopy = pltpu.async_copy(buf, wflat_ref.at[pl.ds(row * y, y)], wsem)
        blk = _UNROLL * _L
        nblk = y // blk
        accs = (jnp.zeros((_L,), jnp.float32),) * _UNROLL

        def body(i, a):
            base = i * blk
            new = []
            for k in range(_UNROLL):
                v = buf[pl.ds(base + k * _L, _L)]
                new.append(a[k] + v * v)
            return tuple(new)

        accs = lax.fori_loop(0, nblk, body, accs)
        ntail = (y - nblk * blk) // _L
        new = list(accs)
        for k in range(ntail):
            v = buf[pl.ds(nblk * blk + k * _L, _L)]
            new[k] = new[k] + v * v
        tot = new[0]
        for k in range(1, _UNROLL):
            tot = tot + new[k]
        s_splat = _lane_sum(tot)
        inv_vec = jnp.where(lane == r, s_splat, inv_vec)
        wcopy.wait()
    outv[...] = _nr_rsqrt(inv_vec)
    pltpu.sync_copy(outv, invp_ref.at[wid, pl.ds(0, _L)])


def _scn_stage(W_y2z):
    Z, Y = W_y2z.shape
    mesh = plsc.VectorSubcoreMesh(core_axis_name="c", subcore_axis_name="s")
    k = pl.kernel(
        functools.partial(_scn_body, y=Y, z=Z),
        out_type=[
            jax.ShapeDtypeStruct((_NW, 128), jnp.float32),
            jax.ShapeDtypeStruct((Z * Y,), jnp.float32),
        ],
        mesh=mesh,
        compiler_params=pltpu.CompilerParams(use_tc_tiling_on_sc=True),
        scratch_types=[
            pltpu.VMEM((Y,), jnp.float32),
            pltpu.VMEM((_L,), jnp.float32),
            pltpu.SemaphoreType.DMA,
            pltpu.SemaphoreType.DMA,
        ],
    )
    return k(W_y2z)


def _scg_body(wflat_ref, idx_ref, invp_ref, out_ref,
              idxv, zoff, fidx, vals, invp_v, invv, sem, *, y, z, bpw):
    wid = lax.axis_index("s") * _NC + lax.axis_index("c")
    base = wid * bpw
    pltpu.sync_copy(idx_ref.at[pl.ds(base, bpw)], idxv)
    pltpu.sync_copy(invp_ref, invp_v)
    lane = lax.broadcasted_iota(jnp.int32, (_L,), 0)
    lm4 = jnp.bitwise_and(lane, 3)
    lg4 = lax.shift_right_logical(lane, 2)
    for k in range(128 // _L):
        acc = jnp.zeros((_L,), jnp.float32)
        for i in range(4):
            r = invp_v[4 * k + i, pl.ds(0, _L)]
            acc = jnp.where(lg4 == i, _lane_shuffle(r, lm4), acc)
        invv[pl.ds(_L * k, _L)] = acc
    for k in range(128 // _L):
        col = lax.broadcasted_iota(jnp.int32, (_L,), 0) + (_L * k)
        zoff[pl.ds(_L * k, _L)] = jnp.where(col < z, col * y, 0)
    chunks = [idxv[pl.ds(c * _L, _L)] for c in range(bpw // _L)]
    for b in range(bpw):
        bidx = chunks[b // _L][b % _L]
        for k in range(128 // _L):
            s = pl.ds(_L * k, _L)
            fidx[b, s] = zoff[s] + bidx
    copies = [
        pltpu.async_copy(wflat_ref.at[fidx.at[b]], vals.at[b], sem)
        for b in range(bpw)
    ]
    for c in copies:
        c.wait()
    for b in range(bpw):
        for k in range(128 // _L):
            s = pl.ds(_L * k, _L)
            vals[b, s] = vals[b, s] * invv[s]
    pltpu.sync_copy(vals, out_ref.at[pl.ds(base, bpw)])


def _scg_stage(wflat, idx, invp, B, Y, Z):
    bpw = B // _NW
    mesh = plsc.VectorSubcoreMesh(core_axis_name="c", subcore_axis_name="s")
    k = pl.kernel(
        functools.partial(_scg_body, y=Y, z=Z, bpw=bpw),
        out_type=jax.ShapeDtypeStruct((B, 128), jnp.float32),
        mesh=mesh,
        compiler_params=pltpu.CompilerParams(use_tc_tiling_on_sc=True),
        scratch_types=[
            pltpu.VMEM((bpw,), jnp.int32),
            pltpu.VMEM((128,), jnp.int32),
            pltpu.VMEM((bpw, 128), jnp.int32),
            pltpu.VMEM((bpw, 128), jnp.float32),
            pltpu.VMEM((_NW, 128), jnp.float32),
            pltpu.VMEM((128,), jnp.float32),
            pltpu.SemaphoreType.DMA,
        ],
    )
    return k(wflat, idx, invp)


def kernel(x, z, W_x2y, W_z2y, W_y2z, per_item):
    B, D = x.shape
    Y = W_x2y.shape[0]
    Z = W_y2z.shape[0]
    idx2d = _tc_stage(x, W_x2y)
    invp, wflat = _scn_stage(W_y2z)
    idx = idx2d.reshape(B)
    outp = _scg_stage(wflat, idx, invp, B, Y, Z)
    out = outp[:, :Z]
    return jnp.where(per_item >= 1, out, jnp.zeros_like(out))
